# Initial kernel scaffold; baseline (speedup 1.0000x reference)
import functools

import jax
import jax.numpy as jnp
from jax import lax
from jax.experimental import pallas as pl
from jax.experimental.pallas import tpu as pltpu
from jax.experimental.pallas import tpu_sc as plsc

VOCAB = 100000
D = 128
B = 16384
NPOS = 10
NNEG = 50

NC = 2
NS = 16
NW = NC * NS
CHUNK = B // NW
G = 8
NG = CHUNK // G
L = 16

_NEG_CHUNKS = ((0, 104), (104, 104), (208, 104), (312, 88))


def _log_sigmoid(x):
    t = jnp.exp(-jnp.abs(x))
    r = t / (t + 2.0)
    r2 = r * r
    p = 1.0 + r2 * (1.0 / 3.0 + r2 * (1.0 / 5.0 + r2 * (1.0 / 7.0 + r2 * (1.0 / 9.0))))
    return jnp.minimum(x, 0.0) - 2.0 * r * p


def _body(in_l, pos_l, neg_l, in_tab, out_tab, out,
          idx_in, idx_p, idx_n, Ri, Rp, Rn, dots, outbuf, sem):
    wid = lax.axis_index("s") * NC + lax.axis_index("c")
    lanes = lax.iota(jnp.int32, L)
    mask_pos = lanes < NPOS
    mask_tail = lanes < 2

    def group(g, _):
        base = wid * CHUNK + g * G
        pltpu.sync_copy(in_l.at[pl.ds(base, G)], idx_in)
        pltpu.sync_copy(pos_l.at[pl.ds(base * NPOS, G * NPOS)], idx_p)
        pltpu.sync_copy(neg_l.at[pl.ds(base * NNEG, G * NNEG)], idx_n)
        copies = [
            pltpu.async_copy(in_tab.at[idx_in], Ri, sem),
            pltpu.async_copy(out_tab.at[idx_p], Rp, sem),
        ]
        for off, ln in _NEG_CHUNKS:
            copies.append(
                pltpu.async_copy(out_tab.at[idx_n.at[pl.ds(off, ln)]],
                                 Rn.at[pl.ds(off, ln)], sem))
        for c in copies:
            c.wait()

        def elem(e, _):
            ie = [Ri[e, pl.ds(k * L, L)] for k in range(8)]

            def dot_rows(rows_ref, row_base, dots_base, n):
                def row(r, _):
                    rr = row_base + r
                    v = rows_ref[rr, pl.ds(0, L)] * ie[0]
                    for k in range(1, 8):
                        v = v + rows_ref[rr, pl.ds(k * L, L)] * ie[k]
                    dots[dots_base + r] = jnp.sum(v)
                    return 0
                lax.fori_loop(0, n, row, 0, unroll=2)

            dot_rows(Rp, e * NPOS, 0, NPOS)
            dot_rows(Rn, e * NNEG, L, NNEG)

            d0 = dots[pl.ds(0, L)]
            acc = jnp.where(mask_pos, _log_sigmoid(d0), 0.0)
            for k in range(1, 4):
                acc = acc + _log_sigmoid(-dots[pl.ds(k * L, L)])
            d4 = dots[pl.ds(4 * L, L)]
            acc = acc + jnp.where(mask_tail, _log_sigmoid(-d4), 0.0)
            outbuf[g * G + e] = -jnp.sum(acc)
            return 0

        lax.fori_loop(0, G, elem, 0)
        return 0

    lax.fori_loop(0, NG, group, 0)
    pltpu.sync_copy(outbuf, out.at[pl.ds(wid * CHUNK, CHUNK)])


_sc_call = functools.partial(
    pl.kernel,
    out_type=jax.ShapeDtypeStruct((B,), jnp.float32),
    mesh=plsc.VectorSubcoreMesh(core_axis_name="c", subcore_axis_name="s",
                                num_cores=NC, num_subcores=NS),
    scratch_types=[
        pltpu.VMEM((G,), jnp.int32),
        pltpu.VMEM((G * NPOS,), jnp.int32),
        pltpu.VMEM((G * NNEG,), jnp.int32),
        pltpu.VMEM((G, D), jnp.float32),
        pltpu.VMEM((G * NPOS, D), jnp.float32),
        pltpu.VMEM((G * NNEG, D), jnp.float32),
        pltpu.VMEM((5 * L,), jnp.float32),
        pltpu.VMEM((CHUNK,), jnp.float32),
        pltpu.SemaphoreType.DMA,
    ],
)(_body)


def kernel(input_labels, positive_labels, negative_labels, input_table,
           output_table):
    pos_flat = positive_labels.reshape(-1)
    neg_flat = negative_labels.reshape(-1)
    return _sc_call(input_labels, pos_flat, neg_flat, input_table,
                    output_table)


# speedup vs baseline: 1.4110x; 1.4110x over previous
import functools

import jax
import jax.numpy as jnp
from jax import lax
from jax.experimental import pallas as pl
from jax.experimental.pallas import tpu as pltpu
from jax.experimental.pallas import tpu_sc as plsc

VOCAB = 100000
D = 128
B = 16384
NPOS = 10
NNEG = 50

NC = 2
NS = 16
NW = NC * NS
CHUNK = B // NW
G = 8
NG = CHUNK // G
L = 16

_NEG_CHUNKS = ((0, 104), (104, 104), (208, 104), (312, 88))


def _log_sigmoid(x):
    t = jnp.exp(-jnp.abs(x))
    r = t / (t + 2.0)
    r2 = r * r
    p = 1.0 + r2 * (1.0 / 3.0 + r2 * (1.0 / 5.0 + r2 * (1.0 / 7.0 + r2 * (1.0 / 9.0))))
    return jnp.minimum(x, 0.0) - 2.0 * r * p


def _vtake(vec, idx):
    dnums = lax.GatherDimensionNumbers(
        offset_dims=(), collapsed_slice_dims=(0,), start_index_map=(0,))
    return lax.gather(vec, idx[:, None], dnums, (1,),
                      mode=lax.GatherScatterMode.PROMISE_IN_BOUNDS)


def _body(in_l, pos_l, neg_l, in_tab, out_tab, out,
          idx_in, idx_p, idx_n, Ri, Rp, Rn, outbuf, sem):
    wid = lax.axis_index("s") * NC + lax.axis_index("c")
    lanes = lax.iota(jnp.int32, L)
    mask_pos = lanes < NPOS
    mask_tail = lanes < 2

    def group(g, _):
        base = wid * CHUNK + g * G
        pltpu.sync_copy(in_l.at[pl.ds(base, G)], idx_in)
        pltpu.sync_copy(pos_l.at[pl.ds(base * NPOS, G * NPOS)], idx_p)
        pltpu.sync_copy(neg_l.at[pl.ds(base * NNEG, G * NNEG)], idx_n)
        copies = [
            pltpu.async_copy(in_tab.at[idx_in], Ri, sem),
            pltpu.async_copy(out_tab.at[idx_p], Rp, sem),
        ]
        for off, ln in _NEG_CHUNKS:
            copies.append(
                pltpu.async_copy(out_tab.at[idx_n.at[pl.ds(off, ln)]],
                                 Rn.at[pl.ds(off, ln)], sem))
        for c in copies:
            c.wait()

        def elem(e, loss_vec):
            rows_p = jnp.minimum(e * NPOS + lanes, G * NPOS - 1)
            rows_n = [jnp.minimum(e * NNEG + k * L + lanes, G * NNEG - 1)
                      for k in range(4)]

            zero = jnp.zeros((L,), jnp.float32)
            accs = [zero] * 5
            for k in range(D // L):
                ie_k = Ri[e, pl.ds(k * L, L)]
                for j in range(L):
                    dcol = jnp.full((L,), k * L + j, jnp.int32)
                    sv = _vtake(ie_k, jnp.full((L,), j, jnp.int32))
                    accs[0] = accs[0] + plsc.load_gather(Rp, [rows_p, dcol]) * sv
                    for m in range(4):
                        accs[m + 1] = (accs[m + 1]
                                       + plsc.load_gather(Rn, [rows_n[m], dcol]) * sv)
            dots = accs

            acc = jnp.where(mask_pos, _log_sigmoid(dots[0]), 0.0)
            for k in range(1, 4):
                acc = acc + _log_sigmoid(-dots[k])
            acc = acc + jnp.where(mask_tail, _log_sigmoid(-dots[4]), 0.0)
            loss = jnp.full((L,), -jnp.sum(acc), jnp.float32)
            return jnp.where(lanes == e, loss, loss_vec)

        loss_vec = lax.fori_loop(0, G, elem, jnp.zeros((L,), jnp.float32))
        plsc.store_scatter(outbuf, [g * G + lanes], loss_vec,
                           mask=lanes < G)
        return 0

    lax.fori_loop(0, NG, group, 0)
    pltpu.sync_copy(outbuf, out.at[pl.ds(wid * CHUNK, CHUNK)])


@functools.cache
def _sc_call():
    return functools.partial(
        pl.kernel,
        out_type=jax.ShapeDtypeStruct((B,), jnp.float32),
        mesh=plsc.VectorSubcoreMesh(core_axis_name="c", subcore_axis_name="s",
                                    num_cores=NC, num_subcores=NS),
        compiler_params=pltpu.CompilerParams(needs_layout_passes=False),
        scratch_types=[
            pltpu.VMEM((G,), jnp.int32),
            pltpu.VMEM((G * NPOS,), jnp.int32),
            pltpu.VMEM((G * NNEG,), jnp.int32),
            pltpu.VMEM((G, D), jnp.float32),
            pltpu.VMEM((G * NPOS, D), jnp.float32),
            pltpu.VMEM((G * NNEG, D), jnp.float32),
            pltpu.VMEM((CHUNK,), jnp.float32),
            pltpu.SemaphoreType.DMA,
        ],
    )(_body)


def kernel(input_labels, positive_labels, negative_labels, input_table,
           output_table):
    pos_flat = positive_labels.reshape(-1)
    neg_flat = negative_labels.reshape(-1)
    return _sc_call()(input_labels, pos_flat, neg_flat, input_table,
                      output_table)


# speedup vs baseline: 5.9025x; 4.1832x over previous
import functools

import jax
import jax.numpy as jnp
from jax import lax
from jax.experimental import pallas as pl
from jax.experimental.pallas import tpu as pltpu
from jax.experimental.pallas import tpu_sc as plsc

VOCAB = 100000
D = 128
B = 16384
NPOS = 10
NNEG = 50

NC = 2
NS = 16
NW = NC * NS
CHUNK = B // NW
G = 8
NG = CHUNK // G
L = 16

_NEG_CHUNKS = ((0, 104), (104, 104), (208, 104), (312, 88))


def _log_sigmoid(x):
    t = jnp.exp(-jnp.abs(x))
    r = t / (t + 2.0)
    r2 = r * r
    p = 1.0 + r2 * (1.0 / 3.0 + r2 * (1.0 / 5.0 + r2 * (1.0 / 7.0 + r2 * (1.0 / 9.0))))
    return jnp.minimum(x, 0.0) - 2.0 * r * p


def _body(in_l, pos_l, neg_l, in_tab, out_tab, out,
          idx_in, idx_p, idx_n, Ri, Rall, outbuf, sem):
    wid = lax.axis_index("s") * NC + lax.axis_index("c")
    lanes = lax.iota(jnp.int32, L)
    sign0 = jnp.where(lanes < NPOS, 1.0, -1.0)
    mask0 = lanes < NPOS + 2

    def group(g, _):
        base = wid * CHUNK + g * G
        pltpu.sync_copy(in_l.at[pl.ds(base, G)], idx_in)
        pltpu.sync_copy(pos_l.at[pl.ds(base * NPOS, G * NPOS)], idx_p)
        pltpu.sync_copy(neg_l.at[pl.ds(base * NNEG, G * NNEG)], idx_n)
        copies = [
            pltpu.async_copy(in_tab.at[idx_in], Ri, sem),
            pltpu.async_copy(out_tab.at[idx_p], Rall.at[pl.ds(0, G * NPOS)],
                             sem),
        ]
        for off, ln in _NEG_CHUNKS:
            copies.append(
                pltpu.async_copy(out_tab.at[idx_n.at[pl.ds(off, ln)]],
                                 Rall.at[pl.ds(G * NPOS + off, ln)], sem))
        for c in copies:
            c.wait()

        def elem(e, loss_vec):
            nbase = G * NPOS + e * NNEG
            rows = [jnp.where(lanes < NPOS, e * NPOS + lanes,
                              jnp.minimum(nbase + 38 + lanes,
                                          G * (NPOS + NNEG) - 1))]
            rows += [nbase + k * L + lanes for k in range(3)]
            esplat = jnp.full((L,), e, jnp.int32)

            zero = jnp.zeros((L,), jnp.float32)
            accs = [zero] * 4
            for dd in range(D):
                dcol = (lanes + dd) & (D - 1)
                sv = plsc.load_gather(Ri, [esplat, dcol])
                for m in range(4):
                    accs[m] = (accs[m]
                               + plsc.load_gather(Rall, [rows[m], dcol]) * sv)
            dots = accs

            acc = jnp.where(mask0, _log_sigmoid(dots[0] * sign0), 0.0)
            for k in range(1, 4):
                acc = acc + _log_sigmoid(-dots[k])
            loss = jnp.full((L,), -jnp.sum(acc), jnp.float32)
            return jnp.where(lanes == e, loss, loss_vec)

        loss_vec = lax.fori_loop(0, G, elem, jnp.zeros((L,), jnp.float32))
        plsc.store_scatter(outbuf, [g * G + lanes], loss_vec,
                           mask=lanes < G)
        return 0

    lax.fori_loop(0, NG, group, 0)
    pltpu.sync_copy(outbuf, out.at[pl.ds(wid * CHUNK, CHUNK)])


@functools.cache
def _sc_call():
    return functools.partial(
        pl.kernel,
        out_type=jax.ShapeDtypeStruct((B,), jnp.float32),
        mesh=plsc.VectorSubcoreMesh(core_axis_name="c", subcore_axis_name="s",
                                    num_cores=NC, num_subcores=NS),
        compiler_params=pltpu.CompilerParams(needs_layout_passes=False),
        scratch_types=[
            pltpu.VMEM((G,), jnp.int32),
            pltpu.VMEM((G * NPOS,), jnp.int32),
            pltpu.VMEM((G * NNEG,), jnp.int32),
            pltpu.VMEM((G, D), jnp.float32),
            pltpu.VMEM((G * (NPOS + NNEG), D), jnp.float32),
            pltpu.VMEM((CHUNK,), jnp.float32),
            pltpu.SemaphoreType.DMA,
        ],
    )(_body)


def kernel(input_labels, positive_labels, negative_labels, input_table,
           output_table):
    pos_flat = positive_labels.reshape(-1)
    neg_flat = negative_labels.reshape(-1)
    return _sc_call()(input_labels, pos_flat, neg_flat, input_table,
                      output_table)


# speedup vs baseline: 8.2325x; 1.3947x over previous
import functools

import jax
import jax.numpy as jnp
from jax import lax
from jax.experimental import pallas as pl
from jax.experimental.pallas import tpu as pltpu
from jax.experimental.pallas import tpu_sc as plsc

VOCAB = 100000
D = 128
B = 16384
NPOS = 10
NNEG = 50

NC = 2
NS = 16
NW = NC * NS
CHUNK = B // NW
G = 8
NG = CHUNK // G
L = 16

_NEG_CHUNKS = ((0, 128), (128, 128), (256, 128), (384, 16))


def _log_sigmoid(x):
    t = jnp.exp(-jnp.abs(x))
    r = t / (t + 2.0)
    r2 = r * r
    p = 1.0 + r2 * (1.0 / 3.0 + r2 * (1.0 / 5.0 + r2 * (1.0 / 7.0 + r2 * (1.0 / 9.0))))
    return jnp.minimum(x, 0.0) - 2.0 * r * p


def _body(in_l, pos_l, neg_l, in_tab, out_tab, out,
          idx_in0, idx_p0, idx_n0, Ri0, Rall0,
          idx_in1, idx_p1, idx_n1, Ri1, Rall1, outbuf, semA, semB):
    wid = lax.axis_index("s") * NC + lax.axis_index("c")
    bufs = ((idx_in0, idx_p0, idx_n0, Ri0, Rall0, semA),
            (idx_in1, idx_p1, idx_n1, Ri1, Rall1, semB))
    lanes = lax.iota(jnp.int32, L)
    sign0 = jnp.where(lanes < NPOS, 1.0, -1.0)
    mask0 = lanes < NPOS + 2

    def copies(g, p, issue):
        base = wid * CHUNK + g * G
        idx_in, idx_p, idx_n, Ri, Rall, sem = bufs[p]
        if issue:
            pltpu.sync_copy(in_l.at[pl.ds(base, G)], idx_in)
            pltpu.sync_copy(pos_l.at[pl.ds(base * NPOS, G * NPOS)], idx_p)
            pltpu.sync_copy(neg_l.at[pl.ds(base * NNEG, G * NNEG)], idx_n)
        mk = pltpu.async_copy if issue else pltpu.make_async_copy
        descs = [
            mk(in_tab.at[idx_in], Ri, sem),
            mk(out_tab.at[idx_p], Rall.at[pl.ds(0, G * NPOS)], sem),
        ]
        for off, ln in _NEG_CHUNKS:
            descs.append(
                mk(out_tab.at[idx_n.at[pl.ds(off, ln)]],
                   Rall.at[pl.ds(G * NPOS + off, ln)], sem))
        return descs

    def compute(g, p):
        Ri_p = bufs[p][3]
        Rall_p = bufs[p][4]

        def elem(e, loss_vec):
            nbase = G * NPOS + e * NNEG
            rows = [jnp.where(lanes < NPOS, e * NPOS + lanes,
                              jnp.minimum(nbase + 38 + lanes,
                                          G * (NPOS + NNEG) - 1))]
            rows += [nbase + k * L + lanes for k in range(3)]
            esplat = jnp.full((L,), e, jnp.int32)

            def dstep(dd, accs):
                dcol = (lanes + dd) & (D - 1)
                sv = plsc.load_gather(Ri_p, [esplat, dcol])
                return tuple(
                    accs[m] + plsc.load_gather(Rall_p, [rows[m], dcol]) * sv
                    for m in range(4))

            zero = jnp.zeros((L,), jnp.float32)
            dots = lax.fori_loop(0, D, dstep, (zero,) * 4, unroll=16)

            acc = jnp.where(mask0, _log_sigmoid(dots[0] * sign0), 0.0)
            for k in range(1, 4):
                acc = acc + _log_sigmoid(-dots[k])
            loss = jnp.full((L,), -jnp.sum(acc), jnp.float32)
            return jnp.where(lanes == e, loss, loss_vec)

        loss_vec = lax.fori_loop(0, G, elem, jnp.zeros((L,), jnp.float32))
        plsc.store_scatter(outbuf, [g * G + lanes], loss_vec,
                           mask=lanes < G)

    def do_group(g, p):
        for desc in copies(g, p, issue=False):
            desc.wait()
        compute(g, p)

        @pl.when(g + 2 < NG)
        def _prefetch():
            copies(g + 2, p, issue=True)

    copies(0, 0, issue=True)
    copies(1, 1, issue=True)

    def pair(i, _):
        do_group(2 * i, 0)
        do_group(2 * i + 1, 1)
        return 0

    lax.fori_loop(0, NG // 2, pair, 0)
    pltpu.sync_copy(outbuf, out.at[pl.ds(wid * CHUNK, CHUNK)])


@functools.cache
def _sc_call():
    return functools.partial(
        pl.kernel,
        out_type=jax.ShapeDtypeStruct((B,), jnp.float32),
        mesh=plsc.VectorSubcoreMesh(core_axis_name="c", subcore_axis_name="s",
                                    num_cores=NC, num_subcores=NS),
        compiler_params=pltpu.CompilerParams(needs_layout_passes=False),
        scratch_types=(
            [pltpu.VMEM((G,), jnp.int32),
             pltpu.VMEM((G * NPOS,), jnp.int32),
             pltpu.VMEM((G * NNEG,), jnp.int32),
             pltpu.VMEM((G, D), jnp.float32),
             pltpu.VMEM((G * (NPOS + NNEG), D), jnp.float32),
             ] * 2
            + [pltpu.VMEM((CHUNK,), jnp.float32),
               pltpu.SemaphoreType.DMA,
               pltpu.SemaphoreType.DMA]),
    )(_body)


def kernel(input_labels, positive_labels, negative_labels, input_table,
           output_table):
    pos_flat = positive_labels.reshape(-1)
    neg_flat = negative_labels.reshape(-1)
    return _sc_call()(input_labels, pos_flat, neg_flat, input_table,
                      output_table)
